# baseline (speedup 1.0000x reference)
import jax
import jax.numpy as jnp
from jax import lax
from jax.experimental import pallas as pl
from jax.experimental.pallas import tpu as pltpu

_B, _C, _H, _W, _D = 2, 64, 32, 32, 32
_WD = _W * _D
_N = _H * _WD
_HT = 8
_M = _HT * _WD
_NT = _N // _M
_MARGIN = 1.0
_W_COMP, _W_SEP, _W_SMOOTH = 1.0, 0.5, 0.3

_NSLOT = 10

_DN = (((1,), (0,)), ((), ()))
_DNT = (((1,), (1,)), ((), ()))


def _rsum(x2d):
    ones = jnp.ones((1, _C), dtype=jnp.bfloat16)
    return lax.dot_general(ones, x2d, _DN, preferred_element_type=jnp.float32)


def _scl_kernel(f_ref, p_ref, t_ref, out_ref, sums, smem, sqs, cf, cc):
    s = pl.program_id(0)
    b = pl.program_id(1)
    i = pl.program_id(2)

    @pl.when((s == 0) & (b == 0) & (i == 0))
    def _init():
        sums[...] = jnp.zeros_like(sums)
        for k in range(_NSLOT):
            smem[k] = 0.0

    f2d = f_ref[0]
    p2 = p_ref[0]
    p1 = jax.nn.sigmoid(p2[1:2] - p2[0:1])
    conf = jnp.maximum(p1, 1.0 - p1)
    m1b = (t_ref[0] == 1)
    m1 = m1b.astype(jnp.float32)

    @pl.when(s == 0)
    def _pass0():
        sq = _rsum(f2d * f2d)
        row = b * _NT + i
        sqs[pl.ds(row, 1), :] = sq

        mstack = jnp.concatenate(
            [jnp.ones((1, _M), jnp.bfloat16), m1b.astype(jnp.bfloat16)], axis=0)
        s2 = lax.dot_general(mstack, f2d, _DNT,
                             preferred_element_type=jnp.float32)
        r = 2 * b
        sums[pl.ds(r, 1), :] = sums[pl.ds(r, 1), :] + (s2[0:1] - s2[1:2])
        sums[pl.ds(r + 1, 1), :] = sums[pl.ds(r + 1, 1), :] + s2[1:2]
        smem[b] = smem[b] + jnp.sum(m1)

        ch = _rsum(f2d[:, _WD:] * f2d[:, :-_WD])
        termh = sq[:, _WD:] + sq[:, :-_WD] - 2.0 * ch
        wh = (conf[:, _WD:] + conf[:, :-_WD]) * 0.5
        acc_h = jnp.sum(termh * wh)

        @pl.when(i > 0)
        def _boundary():
            crossb = _rsum(f2d[:, :_WD] * cf[...])
            sqprev = sqs[pl.ds(row - 1, 1), pl.ds(_M - _WD, _WD)]
            termb = sq[:, :_WD] + sqprev - 2.0 * crossb
            wb = (conf[:, :_WD] + cc[...]) * 0.5
            smem[6] = smem[6] + jnp.sum(termb * wb)

        smem[6] = smem[6] + acc_h
        cf[...] = f2d[:, _M - _WD:]
        cc[...] = conf[:, _M - _WD:]

        cw = _rsum(f2d[:, _D:] * f2d[:, :-_D])
        termw = sq[:, _D:] + sq[:, :-_D] - 2.0 * cw
        ww = (conf[:, _D:] + conf[:, :-_D]) * 0.5
        lanew = lax.broadcasted_iota(jnp.int32, (1, _M - _D), 1)
        validw = (lanew % _WD) < (_WD - _D)
        smem[7] = smem[7] + jnp.sum(jnp.where(validw, termw * ww, 0.0))

        cd = _rsum(f2d[:, 1:] * f2d[:, :-1])
        termd = sq[:, 1:] + sq[:, :-1] - 2.0 * cd
        laned = lax.broadcasted_iota(jnp.int32, (1, _M - 1), 1)
        validd = (laned % _D) != (_D - 1)
        smem[8] = smem[8] + jnp.sum(jnp.where(validd, termd, 0.0))

    @pl.when(s == 1)
    def _pass1():
        n1 = smem[b]
        n0 = jnp.float32(_N) - n1
        r = 2 * b
        c0 = sums[pl.ds(r, 1), :] / n0
        c1 = sums[pl.ds(r + 1, 1), :] / n1
        cs = jnp.concatenate([c0, c1], axis=0)
        cc0 = jnp.sum(c0 * c0)
        cc1 = jnp.sum(c1 * c1)

        row = b * _NT + i
        sq = sqs[pl.ds(row, 1), :]
        dots = lax.dot_general(cs.astype(jnp.bfloat16), f2d, _DN,
                               preferred_element_type=jnp.float32)
        dist0 = jnp.sqrt(jnp.maximum(sq - 2.0 * dots[0:1] + cc0, 0.0))
        dist1 = jnp.sqrt(jnp.maximum(sq - 2.0 * dots[1:2] + cc1, 0.0))
        smem[2 + r] = smem[2 + r] + jnp.sum((1.0 - m1) * dist0 * (1.0 - p1))
        smem[3 + r] = smem[3 + r] + jnp.sum(m1 * dist1 * p1)

        @pl.when(i == 0)
        def _sep():
            dc = c0 - c1
            d01 = jnp.sqrt(jnp.sum(dc * dc))
            smem[9] = smem[9] + jnp.maximum(_MARGIN - d01, 0.0)

    @pl.when((s == 1) & (b == _B - 1) & (i == _NT - 1))
    def _finish():
        comp = jnp.float32(0.0)
        for bb in range(_B):
            n1b = smem[bb]
            n0b = jnp.float32(_N) - n1b
            comp = comp + smem[2 + 2 * bb] / n0b + smem[3 + 2 * bb] / n1b
        comp = comp / jnp.float32(_B * 2)
        sep = smem[9] / jnp.float32(_B)
        denom_hw = jnp.float32(_B * (_H - 1) * _W * _D)
        denom_d = jnp.float32(_B * _C * _H * _W * (_D - 1))
        smooth = smem[6] / denom_hw + smem[7] / denom_hw + 0.1 * smem[8] / denom_d
        out_ref[0, 0] = _W_COMP * comp + _W_SEP * sep + _W_SMOOTH * smooth


@jax.jit
def _run(f, p, t):
    return pl.pallas_call(
        _scl_kernel,
        grid=(2, _B, _NT),
        in_specs=[
            pl.BlockSpec((1, _C, _M), lambda s, b, i: (b, 0, i)),
            pl.BlockSpec((1, 2, _M), lambda s, b, i: (b, 0, i)),
            pl.BlockSpec((1, 1, _M), lambda s, b, i: (b, 0, i)),
        ],
        out_specs=pl.BlockSpec(memory_space=pltpu.SMEM),
        out_shape=jax.ShapeDtypeStruct((1, 1), jnp.float32),
        scratch_shapes=[
            pltpu.VMEM((2 * _B, _C), jnp.float32),
            pltpu.SMEM((_NSLOT,), jnp.float32),
            pltpu.VMEM((_B * _NT, _M), jnp.float32),
            pltpu.VMEM((_C, _WD), jnp.bfloat16),
            pltpu.VMEM((1, _WD), jnp.float32),
        ],
    )(f, p, t)


def kernel(features, predictions, targets):
    f = features.astype(jnp.bfloat16).reshape(_B, _C, _N)
    p = predictions.reshape(_B, 2, _N)
    t = targets.astype(jnp.int32).reshape(_B, 1, _N)
    return _run(f, p, t)[0, 0]


# speedup vs baseline: 1.1540x; 1.1540x over previous
import jax
import jax.numpy as jnp
from jax import lax
from jax.experimental import pallas as pl
from jax.experimental.pallas import tpu as pltpu

_B, _C, _H, _W, _D = 2, 64, 32, 32, 32
_WD = _W * _D
_N = _H * _WD
_HT = 8
_M = _HT * _WD
_NT = _N // _M
_MARGIN = 1.0
_W_COMP, _W_SEP, _W_SMOOTH = 1.0, 0.5, 0.3

_NSLOT = 10

_DN = (((1,), (0,)), ((), ()))
_DNT = (((1,), (1,)), ((), ()))


def _rsum(x2d):
    ones = jnp.ones((1, _C), dtype=jnp.bfloat16)
    return lax.dot_general(ones, x2d, _DN, preferred_element_type=jnp.float32)


def _scl_kernel(f_ref, p_ref, t_ref, out_ref, sums, smem, sqs, cf, cc,
                fvm, mw, md):
    s = pl.program_id(0)
    b = pl.program_id(1)
    i = pl.program_id(2)

    @pl.when((s == 0) & (b == 0) & (i == 0))
    def _init():
        sums[...] = jnp.zeros_like(sums)
        for k in range(_NSLOT):
            smem[k] = 0.0
        lane = lax.broadcasted_iota(jnp.int32, (1, _M), 1)
        mw[...] = jnp.where((lane % _WD) < (_WD - _D), 0.5, 0.0)
        md[...] = jnp.where((lane % _D) != (_D - 1), 1.0, 0.0)

    row = b * _NT + i
    p2 = p_ref[0]
    p1 = jax.nn.sigmoid(p2[1:2] - p2[0:1])
    conf = jnp.maximum(p1, 1.0 - p1)
    m1b = (t_ref[0] == 1)
    m1 = m1b.astype(jnp.float32)

    @pl.when(s == 0)
    def _pass0():
        f2d = f_ref[0].astype(jnp.bfloat16)
        fvm[pl.ds(row, 1)] = f2d[None]

        sq = _rsum(f2d * f2d)
        sqs[pl.ds(row, 1), :] = sq

        mstack = jnp.concatenate(
            [jnp.ones((1, _M), jnp.bfloat16), m1b.astype(jnp.bfloat16)], axis=0)
        s2 = lax.dot_general(mstack, f2d, _DNT,
                             preferred_element_type=jnp.float32)
        r = 2 * b
        sums[pl.ds(r, 1), :] = sums[pl.ds(r, 1), :] + (s2[0:1] - s2[1:2])
        sums[pl.ds(r + 1, 1), :] = sums[pl.ds(r + 1, 1), :] + s2[1:2]
        smem[b] = smem[b] + jnp.sum(m1)

        ch = _rsum(f2d[:, _WD:] * f2d[:, :-_WD])
        termh = sq[:, _WD:] + sq[:, :-_WD] - 2.0 * ch
        wh = (conf[:, _WD:] + conf[:, :-_WD]) * 0.5
        acc_h = jnp.sum(termh * wh)

        @pl.when(i > 0)
        def _boundary():
            crossb = _rsum(f2d[:, :_WD] * cf[...])
            sqprev = sqs[pl.ds(row - 1, 1), pl.ds(_M - _WD, _WD)]
            termb = sq[:, :_WD] + sqprev - 2.0 * crossb
            wb = (conf[:, :_WD] + cc[...]) * 0.5
            smem[6] = smem[6] + jnp.sum(termb * wb)

        smem[6] = smem[6] + acc_h
        cf[...] = f2d[:, _M - _WD:]
        cc[...] = conf[:, _M - _WD:]

        cw = _rsum(f2d[:, _D:] * f2d[:, :-_D])
        termw = sq[:, _D:] + sq[:, :-_D] - 2.0 * cw
        ww = (conf[:, _D:] + conf[:, :-_D]) * mw[:, :_M - _D]
        smem[7] = smem[7] + jnp.sum(termw * ww)

        cd = _rsum(f2d[:, 1:] * f2d[:, :-1])
        termd = sq[:, 1:] + sq[:, :-1] - 2.0 * cd
        smem[8] = smem[8] + jnp.sum(termd * md[:, :_M - 1])

    @pl.when(s == 1)
    def _pass1():
        f2d = fvm[pl.ds(row, 1)][0]
        n1 = smem[b]
        n0 = jnp.float32(_N) - n1
        r = 2 * b
        c0 = sums[pl.ds(r, 1), :] / n0
        c1 = sums[pl.ds(r + 1, 1), :] / n1
        cs = jnp.concatenate([c0, c1], axis=0)
        cc0 = jnp.sum(c0 * c0)
        cc1 = jnp.sum(c1 * c1)

        sq = sqs[pl.ds(row, 1), :]
        dots = lax.dot_general(cs.astype(jnp.bfloat16), f2d, _DN,
                               preferred_element_type=jnp.float32)
        dist0 = jnp.sqrt(jnp.maximum(sq - 2.0 * dots[0:1] + cc0, 0.0))
        dist1 = jnp.sqrt(jnp.maximum(sq - 2.0 * dots[1:2] + cc1, 0.0))
        smem[2 + r] = smem[2 + r] + jnp.sum((1.0 - m1) * dist0 * (1.0 - p1))
        smem[3 + r] = smem[3 + r] + jnp.sum(m1 * dist1 * p1)

        @pl.when(i == 0)
        def _sep():
            dc = c0 - c1
            d01 = jnp.sqrt(jnp.sum(dc * dc))
            smem[9] = smem[9] + jnp.maximum(_MARGIN - d01, 0.0)

    @pl.when((s == 1) & (b == _B - 1) & (i == _NT - 1))
    def _finish():
        comp = jnp.float32(0.0)
        for bb in range(_B):
            n1b = smem[bb]
            n0b = jnp.float32(_N) - n1b
            comp = comp + smem[2 + 2 * bb] / n0b + smem[3 + 2 * bb] / n1b
        comp = comp / jnp.float32(_B * 2)
        sep = smem[9] / jnp.float32(_B)
        denom_hw = jnp.float32(_B * (_H - 1) * _W * _D)
        denom_d = jnp.float32(_B * _C * _H * _W * (_D - 1))
        smooth = smem[6] / denom_hw + smem[7] / denom_hw + 0.1 * smem[8] / denom_d
        out_ref[0, 0] = _W_COMP * comp + _W_SEP * sep + _W_SMOOTH * smooth


@jax.jit
def _run(f, p, t):
    return pl.pallas_call(
        _scl_kernel,
        grid=(2, _B, _NT),
        in_specs=[
            pl.BlockSpec((1, _C, _M),
                         lambda s, b, i: (b * (1 - s), 0, i * (1 - s))),
            pl.BlockSpec((1, 2, _M), lambda s, b, i: (b, 0, i)),
            pl.BlockSpec((1, 1, _M), lambda s, b, i: (b, 0, i)),
        ],
        out_specs=pl.BlockSpec(memory_space=pltpu.SMEM),
        out_shape=jax.ShapeDtypeStruct((1, 1), jnp.float32),
        scratch_shapes=[
            pltpu.VMEM((2 * _B, _C), jnp.float32),
            pltpu.SMEM((_NSLOT,), jnp.float32),
            pltpu.VMEM((_B * _NT, _M), jnp.float32),
            pltpu.VMEM((_C, _WD), jnp.bfloat16),
            pltpu.VMEM((1, _WD), jnp.float32),
            pltpu.VMEM((_B * _NT, _C, _M), jnp.bfloat16),
            pltpu.VMEM((1, _M), jnp.float32),
            pltpu.VMEM((1, _M), jnp.float32),
        ],
    )(f, p, t)


def kernel(features, predictions, targets):
    f = features.reshape(_B, _C, _N)
    p = predictions.reshape(_B, 2, _N)
    t = targets.astype(jnp.int32).reshape(_B, 1, _N)
    return _run(f, p, t)[0, 0]


# speedup vs baseline: 1.2607x; 1.0924x over previous
import jax
import jax.numpy as jnp
from jax import lax
from jax.experimental import pallas as pl
from jax.experimental.pallas import tpu as pltpu

_B, _C, _H, _W, _D = 2, 64, 32, 32, 32
_WD = _W * _D
_N = _H * _WD
_HT = 16
_M = _HT * _WD
_NT = _N // _M
_MARGIN = 1.0
_W_COMP, _W_SEP, _W_SMOOTH = 1.0, 0.5, 0.3

_NSLOT = 10

_DN = (((1,), (0,)), ((), ()))
_DNT = (((1,), (1,)), ((), ()))


def _rsum(x2d):
    ones = jnp.ones((1, _C), dtype=jnp.bfloat16)
    return lax.dot_general(ones, x2d, _DN, preferred_element_type=jnp.float32)


def _scl_kernel(f_ref, p_ref, t_ref, out_ref, sums, smem, sqs, cf, cc,
                fvm, mw, md):
    s = pl.program_id(0)
    b = pl.program_id(1)
    i = pl.program_id(2)

    @pl.when((s == 0) & (b == 0) & (i == 0))
    def _init():
        sums[...] = jnp.zeros_like(sums)
        for k in range(_NSLOT):
            smem[k] = 0.0
        lane = lax.broadcasted_iota(jnp.int32, (1, _M), 1)
        mw[...] = jnp.where((lane % _WD) < (_WD - _D), 0.5, 0.0)
        md[...] = jnp.where((lane % _D) != (_D - 1), 1.0, 0.0)

    row = b * _NT + i
    p2 = p_ref[0]
    p1 = jax.nn.sigmoid(p2[1:2] - p2[0:1])
    conf = jnp.maximum(p1, 1.0 - p1)
    m1b = (t_ref[0] == 1)
    m1 = m1b.astype(jnp.float32)

    @pl.when(s == 0)
    def _pass0():
        f2d = f_ref[0].astype(jnp.bfloat16)
        fvm[pl.ds(row, 1)] = f2d[None]

        sq = _rsum(f2d * f2d)
        sqs[pl.ds(row, 1), :] = sq

        mstack = jnp.concatenate(
            [jnp.ones((1, _M), jnp.bfloat16), m1b.astype(jnp.bfloat16)], axis=0)
        s2 = lax.dot_general(mstack, f2d, _DNT,
                             preferred_element_type=jnp.float32)
        r = 2 * b
        sums[pl.ds(r, 1), :] = sums[pl.ds(r, 1), :] + (s2[0:1] - s2[1:2])
        sums[pl.ds(r + 1, 1), :] = sums[pl.ds(r + 1, 1), :] + s2[1:2]
        smem[b] = smem[b] + jnp.sum(m1)

        ch = _rsum(f2d[:, _WD:] * f2d[:, :-_WD])
        termh = sq[:, _WD:] + sq[:, :-_WD] - 2.0 * ch
        wh = (conf[:, _WD:] + conf[:, :-_WD]) * 0.5
        acc_h = jnp.sum(termh * wh)

        @pl.when(i > 0)
        def _boundary():
            crossb = _rsum(f2d[:, :_WD] * cf[...])
            sqprev = sqs[pl.ds(row - 1, 1), pl.ds(_M - _WD, _WD)]
            termb = sq[:, :_WD] + sqprev - 2.0 * crossb
            wb = (conf[:, :_WD] + cc[...]) * 0.5
            smem[6] = smem[6] + jnp.sum(termb * wb)

        smem[6] = smem[6] + acc_h
        cf[...] = f2d[:, _M - _WD:]
        cc[...] = conf[:, _M - _WD:]

        cw = _rsum(f2d[:, _D:] * f2d[:, :-_D])
        termw = sq[:, _D:] + sq[:, :-_D] - 2.0 * cw
        ww = (conf[:, _D:] + conf[:, :-_D]) * mw[:, :_M - _D]
        smem[7] = smem[7] + jnp.sum(termw * ww)

        cd = _rsum(f2d[:, 1:] * f2d[:, :-1])
        termd = sq[:, 1:] + sq[:, :-1] - 2.0 * cd
        smem[8] = smem[8] + jnp.sum(termd * md[:, :_M - 1])

    @pl.when(s == 1)
    def _pass1():
        f2d = fvm[pl.ds(row, 1)][0]
        n1 = smem[b]
        n0 = jnp.float32(_N) - n1
        r = 2 * b
        c0 = sums[pl.ds(r, 1), :] / n0
        c1 = sums[pl.ds(r + 1, 1), :] / n1
        cs = jnp.concatenate([c0, c1], axis=0)
        cc0 = jnp.sum(c0 * c0)
        cc1 = jnp.sum(c1 * c1)

        sq = sqs[pl.ds(row, 1), :]
        dots = lax.dot_general(cs.astype(jnp.bfloat16), f2d, _DN,
                               preferred_element_type=jnp.float32)
        dist0 = jnp.sqrt(jnp.maximum(sq - 2.0 * dots[0:1] + cc0, 0.0))
        dist1 = jnp.sqrt(jnp.maximum(sq - 2.0 * dots[1:2] + cc1, 0.0))
        smem[2 + r] = smem[2 + r] + jnp.sum((1.0 - m1) * dist0 * (1.0 - p1))
        smem[3 + r] = smem[3 + r] + jnp.sum(m1 * dist1 * p1)

        @pl.when(i == 0)
        def _sep():
            dc = c0 - c1
            d01 = jnp.sqrt(jnp.sum(dc * dc))
            smem[9] = smem[9] + jnp.maximum(_MARGIN - d01, 0.0)

    @pl.when((s == 1) & (b == _B - 1) & (i == _NT - 1))
    def _finish():
        comp = jnp.float32(0.0)
        for bb in range(_B):
            n1b = smem[bb]
            n0b = jnp.float32(_N) - n1b
            comp = comp + smem[2 + 2 * bb] / n0b + smem[3 + 2 * bb] / n1b
        comp = comp / jnp.float32(_B * 2)
        sep = smem[9] / jnp.float32(_B)
        denom_hw = jnp.float32(_B * (_H - 1) * _W * _D)
        denom_d = jnp.float32(_B * _C * _H * _W * (_D - 1))
        smooth = smem[6] / denom_hw + smem[7] / denom_hw + 0.1 * smem[8] / denom_d
        out_ref[0, 0] = _W_COMP * comp + _W_SEP * sep + _W_SMOOTH * smooth


@jax.jit
def _run(f, p, t):
    return pl.pallas_call(
        _scl_kernel,
        grid=(2, _B, _NT),
        in_specs=[
            pl.BlockSpec((1, _C, _M),
                         lambda s, b, i: (b * (1 - s), 0, i * (1 - s))),
            pl.BlockSpec((1, 2, _M), lambda s, b, i: (b, 0, i)),
            pl.BlockSpec((1, 1, _M), lambda s, b, i: (b, 0, i)),
        ],
        out_specs=pl.BlockSpec(memory_space=pltpu.SMEM),
        out_shape=jax.ShapeDtypeStruct((1, 1), jnp.float32),
        scratch_shapes=[
            pltpu.VMEM((2 * _B, _C), jnp.float32),
            pltpu.SMEM((_NSLOT,), jnp.float32),
            pltpu.VMEM((_B * _NT, _M), jnp.float32),
            pltpu.VMEM((_C, _WD), jnp.bfloat16),
            pltpu.VMEM((1, _WD), jnp.float32),
            pltpu.VMEM((_B * _NT, _C, _M), jnp.bfloat16),
            pltpu.VMEM((1, _M), jnp.float32),
            pltpu.VMEM((1, _M), jnp.float32),
        ],
    )(f, p, t)


def kernel(features, predictions, targets):
    f = features.reshape(_B, _C, _N)
    p = predictions.reshape(_B, 2, _N)
    t = targets.astype(jnp.int32).reshape(_B, 1, _N)
    return _run(f, p, t)[0, 0]


# speedup vs baseline: 1.3964x; 1.1077x over previous
import jax
import jax.numpy as jnp
from jax import lax
from jax.experimental import pallas as pl
from jax.experimental.pallas import tpu as pltpu

_B, _C, _H, _W, _D = 2, 64, 32, 32, 32
_WD = _W * _D
_N = _H * _WD
_HT = 32
_M = _HT * _WD
_NT = _N // _M
_MARGIN = 1.0
_W_COMP, _W_SEP, _W_SMOOTH = 1.0, 0.5, 0.3

_NSLOT = 10

_DN = (((1,), (0,)), ((), ()))
_DNT = (((1,), (1,)), ((), ()))


def _rsum(x2d):
    ones = jnp.ones((1, _C), dtype=jnp.bfloat16)
    return lax.dot_general(ones, x2d, _DN, preferred_element_type=jnp.float32)


def _scl_kernel(f_ref, p_ref, t_ref, out_ref, sums, smem, sqs, cf, cc,
                fvm, mw, md):
    s = pl.program_id(0)
    b = pl.program_id(1)
    i = pl.program_id(2)

    @pl.when((s == 0) & (b == 0) & (i == 0))
    def _init():
        sums[...] = jnp.zeros_like(sums)
        for k in range(_NSLOT):
            smem[k] = 0.0
        lane = lax.broadcasted_iota(jnp.int32, (1, _M), 1)
        mw[...] = jnp.where((lane % _WD) < (_WD - _D), 0.5, 0.0)
        md[...] = jnp.where((lane % _D) != (_D - 1), 1.0, 0.0)

    row = b * _NT + i
    p2 = p_ref[0]
    p1 = jax.nn.sigmoid(p2[1:2] - p2[0:1])
    conf = jnp.maximum(p1, 1.0 - p1)
    m1b = (t_ref[0] == 1)
    m1 = m1b.astype(jnp.float32)

    @pl.when(s == 0)
    def _pass0():
        f2d = f_ref[0].astype(jnp.bfloat16)
        fvm[pl.ds(row, 1)] = f2d[None]

        sq = _rsum(f2d * f2d)
        sqs[pl.ds(row, 1), :] = sq

        mstack = jnp.concatenate(
            [jnp.ones((1, _M), jnp.bfloat16), m1b.astype(jnp.bfloat16)], axis=0)
        s2 = lax.dot_general(mstack, f2d, _DNT,
                             preferred_element_type=jnp.float32)
        r = 2 * b
        sums[pl.ds(r, 1), :] = sums[pl.ds(r, 1), :] + (s2[0:1] - s2[1:2])
        sums[pl.ds(r + 1, 1), :] = sums[pl.ds(r + 1, 1), :] + s2[1:2]
        smem[b] = smem[b] + jnp.sum(m1)

        ch = _rsum(f2d[:, _WD:] * f2d[:, :-_WD])
        termh = sq[:, _WD:] + sq[:, :-_WD] - 2.0 * ch
        wh = (conf[:, _WD:] + conf[:, :-_WD]) * 0.5
        acc_h = jnp.sum(termh * wh)

        @pl.when(i > 0)
        def _boundary():
            crossb = _rsum(f2d[:, :_WD] * cf[...])
            sqprev = sqs[pl.ds(row - 1, 1), pl.ds(_M - _WD, _WD)]
            termb = sq[:, :_WD] + sqprev - 2.0 * crossb
            wb = (conf[:, :_WD] + cc[...]) * 0.5
            smem[6] = smem[6] + jnp.sum(termb * wb)

        smem[6] = smem[6] + acc_h
        cf[...] = f2d[:, _M - _WD:]
        cc[...] = conf[:, _M - _WD:]

        cw = _rsum(f2d[:, _D:] * f2d[:, :-_D])
        termw = sq[:, _D:] + sq[:, :-_D] - 2.0 * cw
        ww = (conf[:, _D:] + conf[:, :-_D]) * mw[:, :_M - _D]
        smem[7] = smem[7] + jnp.sum(termw * ww)

        cd = _rsum(f2d[:, 1:] * f2d[:, :-1])
        termd = sq[:, 1:] + sq[:, :-1] - 2.0 * cd
        smem[8] = smem[8] + jnp.sum(termd * md[:, :_M - 1])

    @pl.when(s == 1)
    def _pass1():
        f2d = fvm[pl.ds(row, 1)][0]
        n1 = smem[b]
        n0 = jnp.float32(_N) - n1
        r = 2 * b
        c0 = sums[pl.ds(r, 1), :] / n0
        c1 = sums[pl.ds(r + 1, 1), :] / n1
        cs = jnp.concatenate([c0, c1], axis=0)
        cc0 = jnp.sum(c0 * c0)
        cc1 = jnp.sum(c1 * c1)

        sq = sqs[pl.ds(row, 1), :]
        dots = lax.dot_general(cs.astype(jnp.bfloat16), f2d, _DN,
                               preferred_element_type=jnp.float32)
        dist0 = jnp.sqrt(jnp.maximum(sq - 2.0 * dots[0:1] + cc0, 0.0))
        dist1 = jnp.sqrt(jnp.maximum(sq - 2.0 * dots[1:2] + cc1, 0.0))
        smem[2 + r] = smem[2 + r] + jnp.sum((1.0 - m1) * dist0 * (1.0 - p1))
        smem[3 + r] = smem[3 + r] + jnp.sum(m1 * dist1 * p1)

        @pl.when(i == 0)
        def _sep():
            dc = c0 - c1
            d01 = jnp.sqrt(jnp.sum(dc * dc))
            smem[9] = smem[9] + jnp.maximum(_MARGIN - d01, 0.0)

    @pl.when((s == 1) & (b == _B - 1) & (i == _NT - 1))
    def _finish():
        comp = jnp.float32(0.0)
        for bb in range(_B):
            n1b = smem[bb]
            n0b = jnp.float32(_N) - n1b
            comp = comp + smem[2 + 2 * bb] / n0b + smem[3 + 2 * bb] / n1b
        comp = comp / jnp.float32(_B * 2)
        sep = smem[9] / jnp.float32(_B)
        denom_hw = jnp.float32(_B * (_H - 1) * _W * _D)
        denom_d = jnp.float32(_B * _C * _H * _W * (_D - 1))
        smooth = smem[6] / denom_hw + smem[7] / denom_hw + 0.1 * smem[8] / denom_d
        out_ref[0, 0] = _W_COMP * comp + _W_SEP * sep + _W_SMOOTH * smooth


@jax.jit
def _run(f, p, t):
    return pl.pallas_call(
        _scl_kernel,
        grid=(2, _B, _NT),
        in_specs=[
            pl.BlockSpec((1, _C, _M),
                         lambda s, b, i: (b * (1 - s), 0, i * (1 - s))),
            pl.BlockSpec((1, 2, _M), lambda s, b, i: (b, 0, i)),
            pl.BlockSpec((1, 1, _M), lambda s, b, i: (b, 0, i)),
        ],
        out_specs=pl.BlockSpec(memory_space=pltpu.SMEM),
        out_shape=jax.ShapeDtypeStruct((1, 1), jnp.float32),
        scratch_shapes=[
            pltpu.VMEM((2 * _B, _C), jnp.float32),
            pltpu.SMEM((_NSLOT,), jnp.float32),
            pltpu.VMEM((_B * _NT, _M), jnp.float32),
            pltpu.VMEM((_C, _WD), jnp.bfloat16),
            pltpu.VMEM((1, _WD), jnp.float32),
            pltpu.VMEM((_B * _NT, _C, _M), jnp.bfloat16),
            pltpu.VMEM((1, _M), jnp.float32),
            pltpu.VMEM((1, _M), jnp.float32),
        ],
    )(f, p, t)


def kernel(features, predictions, targets):
    f = features.reshape(_B, _C, _N)
    p = predictions.reshape(_B, 2, _N)
    t = targets.astype(jnp.int32).reshape(_B, 1, _N)
    return _run(f, p, t)[0, 0]


# speedup vs baseline: 1.5503x; 1.1102x over previous
import jax
import jax.numpy as jnp
from jax import lax
from jax.experimental import pallas as pl
from jax.experimental.pallas import tpu as pltpu

_B, _C, _H, _W, _D = 2, 64, 32, 32, 32
_WD = _W * _D
_N = _H * _WD
_N8 = _N // 8
_MARGIN = 1.0
_W_COMP, _W_SEP, _W_SMOOTH = 1.0, 0.5, 0.3

_NSLOT = 10

_DN = (((1,), (0,)), ((), ()))
_DNT = (((1,), (1,)), ((), ()))


def _rsum(x2d):
    ones = jnp.ones((1, _C), dtype=jnp.bfloat16)
    return lax.dot_general(ones, x2d, _DN, preferred_element_type=jnp.float32)


def _shifted(x, d):
    xs = jnp.concatenate([x[1:], x[:1]], axis=0)
    return jnp.concatenate([x[:, d:], xs[:, :d]], axis=1)


def _roll_lanes(x, d):
    return jnp.concatenate([x[:, d:], x[:, :d]], axis=1)


def _scl_kernel(f_ref, p_ref, tu_ref, tp_ref, out_ref, sums, smem, sqs,
                fvm, mh, mw, md):
    s = pl.program_id(0)
    b = pl.program_id(1)

    @pl.when((s == 0) & (b == 0))
    def _init():
        sums[...] = jnp.zeros_like(sums)
        for k in range(_NSLOT):
            smem[k] = 0.0
        kk = (lax.broadcasted_iota(jnp.int32, (8, _N8), 0) * _N8
              + lax.broadcasted_iota(jnp.int32, (8, _N8), 1))
        mh[...] = jnp.where(kk < (_N - _WD), 0.5, 0.0)
        mw[...] = jnp.where((kk % _WD) < (_WD - _D), 0.5, 0.0)
        md[...] = jnp.where((kk % _D) != (_D - 1), 1.0, 0.0)

    p2 = p_ref[0]
    p1 = jax.nn.sigmoid(p2[8:16] - p2[0:8])
    conf = jnp.maximum(p1, 1.0 - p1)
    m1 = (tp_ref[0] == 1).astype(jnp.float32)
    r = 2 * b

    @pl.when(s == 0)
    def _pass0():
        f2d = f_ref[0].astype(jnp.bfloat16)
        fvm[pl.ds(b, 1)] = f2d[None]

        sq1 = _rsum(f2d * f2d)
        ch1 = _rsum(f2d * _roll_lanes(f2d, _WD))
        cw1 = _rsum(f2d * _roll_lanes(f2d, _D))
        cd1 = _rsum(f2d * _roll_lanes(f2d, 1))
        R = jnp.concatenate([sq1, ch1, cw1, cd1], axis=0).reshape(32, _N8)
        sq, ch, cw, cd = R[0:8], R[8:16], R[16:24], R[24:32]
        sqs[pl.ds(8 * b, 8)] = sq

        m1u = (tu_ref[0] == 1)
        mstack = jnp.concatenate(
            [jnp.ones((1, _N), jnp.bfloat16), m1u.astype(jnp.bfloat16)], axis=0)
        s2 = lax.dot_general(mstack, f2d, _DNT,
                             preferred_element_type=jnp.float32)
        sums[pl.ds(r, 1), :] = sums[pl.ds(r, 1), :] + (s2[0:1] - s2[1:2])
        sums[pl.ds(r + 1, 1), :] = sums[pl.ds(r + 1, 1), :] + s2[1:2]
        smem[b] = smem[b] + jnp.sum(m1)

        termh = sq + _shifted(sq, _WD) - 2.0 * ch
        smem[6] = smem[6] + jnp.sum(termh * ((conf + _shifted(conf, _WD))
                                             * mh[...]))
        termw = sq + _shifted(sq, _D) - 2.0 * cw
        smem[7] = smem[7] + jnp.sum(termw * ((conf + _shifted(conf, _D))
                                             * mw[...]))
        termd = sq + _shifted(sq, 1) - 2.0 * cd
        smem[8] = smem[8] + jnp.sum(termd * md[...])

    @pl.when(s == 1)
    def _pass1():
        f2d = fvm[pl.ds(b, 1)][0]
        n1 = smem[b]
        n0 = jnp.float32(_N) - n1
        c0 = sums[pl.ds(r, 1), :] / n0
        c1 = sums[pl.ds(r + 1, 1), :] / n1
        cs = jnp.concatenate([c0, c1], axis=0)
        cc0 = jnp.sum(c0 * c0)
        cc1 = jnp.sum(c1 * c1)

        sq = sqs[pl.ds(8 * b, 8)]
        dots = lax.dot_general(cs.astype(jnp.bfloat16), f2d, _DN,
                               preferred_element_type=jnp.float32)
        Rd = dots.reshape(16, _N8)
        dist0 = jnp.sqrt(jnp.maximum(sq - 2.0 * Rd[0:8] + cc0, 0.0))
        dist1 = jnp.sqrt(jnp.maximum(sq - 2.0 * Rd[8:16] + cc1, 0.0))
        smem[2 + r] = smem[2 + r] + jnp.sum((1.0 - m1) * dist0 * (1.0 - p1))
        smem[3 + r] = smem[3 + r] + jnp.sum(m1 * dist1 * p1)

        dc = c0 - c1
        d01 = jnp.sqrt(jnp.sum(dc * dc))
        smem[9] = smem[9] + jnp.maximum(_MARGIN - d01, 0.0)

    @pl.when((s == 1) & (b == _B - 1))
    def _finish():
        comp = jnp.float32(0.0)
        for bb in range(_B):
            n1b = smem[bb]
            n0b = jnp.float32(_N) - n1b
            comp = comp + smem[2 + 2 * bb] / n0b + smem[3 + 2 * bb] / n1b
        comp = comp / jnp.float32(_B * 2)
        sep = smem[9] / jnp.float32(_B)
        denom_hw = jnp.float32(_B * (_H - 1) * _W * _D)
        denom_d = jnp.float32(_B * _C * _H * _W * (_D - 1))
        smooth = smem[6] / denom_hw + smem[7] / denom_hw + 0.1 * smem[8] / denom_d
        out_ref[0, 0] = _W_COMP * comp + _W_SEP * sep + _W_SMOOTH * smooth


@jax.jit
def _run(f, p, tu, tp):
    return pl.pallas_call(
        _scl_kernel,
        grid=(2, _B),
        in_specs=[
            pl.BlockSpec((1, _C, _N), lambda s, b: (b * (1 - s) + s, 0, 0)),
            pl.BlockSpec((1, 16, _N8), lambda s, b: (b, 0, 0)),
            pl.BlockSpec((1, 1, _N), lambda s, b: (b, 0, 0)),
            pl.BlockSpec((1, 8, _N8), lambda s, b: (b, 0, 0)),
        ],
        out_specs=pl.BlockSpec(memory_space=pltpu.SMEM),
        out_shape=jax.ShapeDtypeStruct((1, 1), jnp.float32),
        scratch_shapes=[
            pltpu.VMEM((2 * _B, _C), jnp.float32),
            pltpu.SMEM((_NSLOT,), jnp.float32),
            pltpu.VMEM((8 * _B, _N8), jnp.float32),
            pltpu.VMEM((_B, _C, _N), jnp.bfloat16),
            pltpu.VMEM((8, _N8), jnp.float32),
            pltpu.VMEM((8, _N8), jnp.float32),
            pltpu.VMEM((8, _N8), jnp.float32),
        ],
    )(f, p, tu, tp)


def kernel(features, predictions, targets):
    f = features.reshape(_B, _C, _N)
    p = predictions.reshape(_B, 16, _N8)
    t = targets.astype(jnp.int32)
    tu = t.reshape(_B, 1, _N)
    tp = t.reshape(_B, 8, _N8)
    return _run(f, p, tu, tp)[0, 0]
